# baseline (speedup 1.0000x reference)
import functools
import math

import jax
import jax.numpy as jnp
from jax import lax
from jax.experimental import pallas as pl
from jax.experimental.pallas import tpu as pltpu
from jax.experimental.pallas import tpu_sc as plsc

B = 8
N = 2048
KNB = 20
R = 256
TOT = B * N * KNB
NW = 32
CHUNK = TOT // NW
NTAB = B * N
NEG = -3.0e38


def _sc_gather3(tx, ty, tz, idx_flat):
    mesh = plsc.VectorSubcoreMesh(core_axis_name="c", subcore_axis_name="s")

    @functools.partial(
        pl.kernel,
        mesh=mesh,
        compiler_params=pltpu.CompilerParams(needs_layout_passes=False),
        out_type=[jax.ShapeDtypeStruct((TOT,), jnp.float32)] * 3,
        scratch_types=[
            pltpu.VMEM((NTAB,), jnp.float32),
            pltpu.VMEM((NTAB,), jnp.float32),
            pltpu.VMEM((NTAB,), jnp.float32),
            pltpu.VMEM((CHUNK,), jnp.int32),
            pltpu.VMEM((CHUNK,), jnp.float32),
            pltpu.VMEM((CHUNK,), jnp.float32),
            pltpu.VMEM((CHUNK,), jnp.float32),
        ],
    )
    def gk(tx_h, ty_h, tz_h, idx_h, ox_h, oy_h, oz_h,
           tx_v, ty_v, tz_v, idx_v, ox_v, oy_v, oz_v):
        wid = lax.axis_index("s") * 2 + lax.axis_index("c")
        base = wid * CHUNK
        pltpu.sync_copy(tx_h, tx_v)
        pltpu.sync_copy(ty_h, ty_v)
        pltpu.sync_copy(tz_h, tz_v)
        pltpu.sync_copy(idx_h.at[pl.ds(base, CHUNK)], idx_v)

        def body(i, _):
            iv = idx_v[pl.ds(i * 16, 16)]
            ox_v[pl.ds(i * 16, 16)] = plsc.load_gather(tx_v, [iv])
            oy_v[pl.ds(i * 16, 16)] = plsc.load_gather(ty_v, [iv])
            oz_v[pl.ds(i * 16, 16)] = plsc.load_gather(tz_v, [iv])
            return 0

        lax.fori_loop(0, CHUNK // 16, body, 0)
        pltpu.sync_copy(ox_v, ox_h.at[pl.ds(base, CHUNK)])
        pltpu.sync_copy(oy_v, oy_h.at[pl.ds(base, CHUNK)])
        pltpu.sync_copy(oz_v, oz_h.at[pl.ds(base, CHUNK)])

    return gk(tx, ty, tz, idx_flat)


def _gather3(tx, ty, tz, idx):
    ox, oy, oz = _sc_gather3(tx, ty, tz, idx.reshape(-1))
    return (ox.reshape(B, N, KNB), oy.reshape(B, N, KNB),
            oz.reshape(B, N, KNB))


def _q(v):
    b = jax.lax.bitcast_convert_type(v, jnp.uint32)
    lsb = (b >> jnp.uint32(16)) & jnp.uint32(1)
    r = (b + jnp.uint32(0x7FFF) + lsb) & jnp.uint32(0xFFFF0000)
    return jax.lax.bitcast_convert_type(r, jnp.float32)


def _knn_body(cols_ref, rows_ref, out_ref):
    b = pl.program_id(0)
    rows = rows_ref[0]
    x0 = cols_ref[0, 0:1, :]
    x1 = cols_ref[0, 1:2, :]
    x2 = cols_ref[0, 2:3, :]
    r0 = rows[:, 0:1]
    r1 = rows[:, 1:2]
    r2 = rows[:, 2:3]
    rr = r0 * r0 + r1 * r1 + r2 * r2
    cc = x0 * x0 + x1 * x1 + x2 * x2
    dot = (_q(r0) * _q(x0) + _q(r1) * _q(x1) + _q(r2) * _q(x2))
    inner = -2.0 * dot
    s = (-rr - inner) - cc
    iota = lax.broadcasted_iota(jnp.int32, (R, N), 1)
    for j in range(KNB):
        m = jnp.max(s, axis=1, keepdims=True)
        hit = s >= m
        idx = jnp.min(jnp.where(hit, iota, N), axis=1, keepdims=True)
        out_ref[0, :, j:j + 1] = idx + b * N
        s = jnp.where(iota == idx, NEG, s)


def _knn(xp, xt):
    return pl.pallas_call(
        _knn_body,
        grid=(B, N // R),
        in_specs=[
            pl.BlockSpec((1, 3, N), lambda b, i: (b, 0, 0)),
            pl.BlockSpec((1, R, 3), lambda b, i: (b, i, 0)),
        ],
        out_specs=pl.BlockSpec((1, R, KNB), lambda b, i: (b, i, 0)),
        out_shape=jax.ShapeDtypeStruct((B, N, KNB), jnp.int32),
        interpret=False,
    )(xp, xt)


def _acos(x):
    ax = jnp.abs(x)
    t = jnp.sqrt(jnp.maximum(1.0 - ax, 0.0))
    p = t * (1.5707288 + ax * (-0.2121144 + ax * (0.0742610 + ax * (-0.0187293))))
    return jnp.where(x >= 0.0, p, math.pi - p)


def _cov_eig_body(nx_ref, ny_ref, nz_ref, xt_ref, out_ref):
    pt = xt_ref[0]
    dx = _q(nx_ref[0] - pt[:, 0:1])
    dy = _q(ny_ref[0] - pt[:, 1:2])
    dz = _q(nz_ref[0] - pt[:, 2:3])
    a00 = jnp.sum(dx * dx, axis=1, keepdims=True)
    a11 = jnp.sum(dy * dy, axis=1, keepdims=True)
    a22 = jnp.sum(dz * dz, axis=1, keepdims=True)
    a01 = jnp.sum(dx * dy, axis=1, keepdims=True)
    a02 = jnp.sum(dx * dz, axis=1, keepdims=True)
    a12 = jnp.sum(dy * dz, axis=1, keepdims=True)

    q = (a00 + a11 + a22) / 3.0
    p1 = a01 * a01 + a02 * a02 + a12 * a12
    b00 = a00 - q
    b11 = a11 - q
    b22 = a22 - q
    p2 = b00 * b00 + b11 * b11 + b22 * b22 + 2.0 * p1
    p = jnp.sqrt(jnp.maximum(p2 / 6.0, 0.0))
    pinv = jnp.where(p > 1e-20, 1.0 / jnp.maximum(p, 1e-30), 0.0)
    c00 = b00 * pinv
    c11 = b11 * pinv
    c22 = b22 * pinv
    c01 = a01 * pinv
    c02 = a02 * pinv
    c12 = a12 * pinv
    det = (c00 * (c11 * c22 - c12 * c12)
           - c01 * (c01 * c22 - c12 * c02)
           + c02 * (c01 * c12 - c11 * c02))
    r = jnp.clip(det * 0.5, -1.0, 1.0)
    phi = _acos(r) / 3.0
    c1 = jnp.cos(phi)
    c3 = jnp.cos(phi + 2.0 * math.pi / 3.0)
    for c_name in range(2):
        d1 = 12.0 * c1 * c1 - 3.0
        ok1 = jnp.abs(d1) > 1e-3
        c1 = c1 - jnp.where(
            ok1, (4.0 * c1 * c1 * c1 - 3.0 * c1 - r)
            / jnp.where(ok1, d1, 1.0), 0.0)
        d3 = 12.0 * c3 * c3 - 3.0
        ok3 = jnp.abs(d3) > 1e-3
        c3 = c3 - jnp.where(
            ok3, (4.0 * c3 * c3 * c3 - 3.0 * c3 - r)
            / jnp.where(ok3, d3, 1.0), 0.0)
    e1 = q + 2.0 * p * c1
    e3 = q + 2.0 * p * c3
    e2 = 3.0 * q - e1 - e3
    out_ref[0, :, 0:1] = e3
    out_ref[0, :, 1:2] = e2
    out_ref[0, :, 2:3] = e1


def _cov_eig(nx, ny, nz, xt):
    return pl.pallas_call(
        _cov_eig_body,
        grid=(B, N // R),
        in_specs=[
            pl.BlockSpec((1, R, KNB), lambda b, i: (b, i, 0)),
            pl.BlockSpec((1, R, KNB), lambda b, i: (b, i, 0)),
            pl.BlockSpec((1, R, KNB), lambda b, i: (b, i, 0)),
            pl.BlockSpec((1, R, 3), lambda b, i: (b, i, 0)),
        ],
        out_specs=pl.BlockSpec((1, R, 3), lambda b, i: (b, i, 0)),
        out_shape=jax.ShapeDtypeStruct((B, N, 3), jnp.float32),
        interpret=False,
    )(nx, ny, nz, xt)


def _feat(j, nx, ny, nz, fx, fy, fz, pt, ev):
    dx = nx[:, j:j + 1] - pt[:, 0:1]
    dy = ny[:, j:j + 1] - pt[:, 1:2]
    dz = nz[:, j:j + 1] - pt[:, 2:3]
    gx = fx[:, j:j + 1] - ev[:, 0:1]
    gy = fy[:, j:j + 1] - ev[:, 1:2]
    gz = fz[:, j:j + 1] - ev[:, 2:3]
    dist = jnp.sqrt(dx * dx + dy * dy + dz * dz + 1e-12)
    return jnp.concatenate(
        [dx, dy, dz,
         nx[:, j:j + 1], ny[:, j:j + 1], nz[:, j:j + 1],
         gx, gy, gz,
         fx[:, j:j + 1], fy[:, j:j + 1], fz[:, j:j + 1],
         dist], axis=1)


def _mom_body(nx_ref, ny_ref, nz_ref, fx_ref, fy_ref, fz_ref,
              xt_ref, ev_ref, g_ref):
    b = pl.program_id(0)
    i = pl.program_id(1)

    @pl.when(jnp.logical_and(b == 0, i == 0))
    def _():
        g_ref[...] = jnp.zeros((16, 16), jnp.float32)

    nx = nx_ref[0]
    ny = ny_ref[0]
    nz = nz_ref[0]
    fx = fx_ref[0]
    fy = fy_ref[0]
    fz = fz_ref[0]
    pt = xt_ref[0]
    ev = ev_ref[0]
    acc = jnp.zeros((16, 16), jnp.float32)
    ones = jnp.ones((R, 1), jnp.float32)
    zeros = jnp.zeros((R, 2), jnp.float32)
    for j in range(KNB):
        f = _q(_feat(j, nx, ny, nz, fx, fy, fz, pt, ev))
        f16 = jnp.concatenate([f, ones, zeros], axis=1)
        acc = acc + lax.dot_general(
            f16, f16, (((0,), (0,)), ((), ())),
            preferred_element_type=jnp.float32)
    g_ref[...] += acc


def _moments(nx, ny, nz, fx, fy, fz, xt, ev):
    plane = pl.BlockSpec((1, R, KNB), lambda b, i: (b, i, 0))
    three = pl.BlockSpec((1, R, 3), lambda b, i: (b, i, 0))
    return pl.pallas_call(
        _mom_body,
        grid=(B, N // R),
        in_specs=[plane, plane, plane, plane, plane, plane, three, three],
        out_specs=pl.BlockSpec((16, 16), lambda b, i: (0, 0)),
        out_shape=jax.ShapeDtypeStruct((16, 16), jnp.float32),
        interpret=False,
    )(nx, ny, nz, fx, fy, fz, xt, ev)


def _final_body(nx_ref, ny_ref, nz_ref, fx_ref, fy_ref, fz_ref,
                xt_ref, ev_ref, g_ref, wt_ref, gm_ref, bt_ref, out_ref):
    g = g_ref[...]
    m1 = g[13:14, 0:13]
    cnt = g[13:14, 13:14]
    wt = _q(wt_ref[...])
    gm = gm_ref[...]
    bt = bt_ref[...]
    inv_cnt = 1.0 / cnt[0, 0]
    mu_f = m1 * inv_cnt
    m2 = g[0:13, 0:13] * inv_cnt
    mean_y = lax.dot_general(mu_f, wt, (((1,), (0,)), ((), ())),
                             preferred_element_type=jnp.float32)
    b1 = lax.dot_general(m2, wt, (((1,), (0,)), ((), ())),
                         preferred_element_type=jnp.float32)
    e2 = jnp.sum(wt * b1, axis=0, keepdims=True)
    var = jnp.maximum(e2 - mean_y * mean_y, 0.0)
    scale = gm / jnp.sqrt(var + 1e-5)
    shift = bt - mean_y * scale

    nx = nx_ref[0]
    ny = ny_ref[0]
    nz = nz_ref[0]
    fx = fx_ref[0]
    fy = fy_ref[0]
    fz = fz_ref[0]
    pt = xt_ref[0]
    ev = ev_ref[0]
    acc = jnp.full((R, 64), NEG, jnp.float32)
    for j in range(KNB):
        f = _q(_feat(j, nx, ny, nz, fx, fy, fz, pt, ev))
        y = lax.dot_general(f, wt, (((1,), (0,)), ((), ())),
                            preferred_element_type=jnp.float32)
        y = y * scale + shift
        y = jnp.where(y >= 0.0, y, 0.2 * y)
        acc = jnp.maximum(acc, y)
    out_ref[0] = acc


def _final(nx, ny, nz, fx, fy, fz, xt, ev, g, wt, gm, bt):
    plane = pl.BlockSpec((1, R, KNB), lambda b, i: (b, i, 0))
    three = pl.BlockSpec((1, R, 3), lambda b, i: (b, i, 0))
    const2 = lambda shape: pl.BlockSpec(shape, lambda b, i: (0, 0))
    return pl.pallas_call(
        _final_body,
        grid=(B, N // R),
        in_specs=[plane, plane, plane, plane, plane, plane, three, three,
                  const2((16, 16)), const2((13, 64)),
                  const2((1, 64)), const2((1, 64))],
        out_specs=pl.BlockSpec((1, R, 64), lambda b, i: (b, i, 0)),
        out_shape=jax.ShapeDtypeStruct((B, N, 64), jnp.float32),
        interpret=False,
    )(nx, ny, nz, fx, fy, fz, xt, ev, g, wt, gm, bt)


def kernel(x, W, gamma, beta):
    xt = jnp.transpose(x, (0, 2, 1))
    tx = x[:, 0, :].reshape(-1)
    ty = x[:, 1, :].reshape(-1)
    tz = x[:, 2, :].reshape(-1)

    idx_eu = _knn(x, xt)
    nx, ny, nz = _gather3(tx, ty, tz, idx_eu)

    ev = _cov_eig(nx, ny, nz, xt)
    evp = jnp.transpose(ev, (0, 2, 1))
    idx_ei = _knn(evp, ev)
    fx, fy, fz = _gather3(ev[:, :, 0].reshape(-1),
                          ev[:, :, 1].reshape(-1),
                          ev[:, :, 2].reshape(-1), idx_ei)

    g = _moments(nx, ny, nz, fx, fy, fz, xt, ev)
    out = _final(nx, ny, nz, fx, fy, fz, xt, ev, g,
                 W.T, gamma.reshape(1, 64), beta.reshape(1, 64))
    return jnp.transpose(out, (0, 2, 1))


# speedup vs baseline: 1.9522x; 1.9522x over previous
import functools
import math

import jax
import jax.numpy as jnp
from jax import lax
from jax.experimental import pallas as pl
from jax.experimental.pallas import tpu as pltpu
from jax.experimental.pallas import tpu_sc as plsc

B = 8
N = 2048
KNB = 20
R = 256
TN = 512
TOT = B * N * KNB
NW = 32
CHUNK = TOT // NW
NTAB = B * N
NEG = -3.0e38


def _q(v):
    b = jax.lax.bitcast_convert_type(v, jnp.uint32)
    lsb = (b >> jnp.uint32(16)) & jnp.uint32(1)
    r = (b + jnp.uint32(0x7FFF) + lsb) & jnp.uint32(0xFFFF0000)
    return jax.lax.bitcast_convert_type(r, jnp.float32)


def _sc_gather6(tx, ty, tz, idx_flat):
    mesh = plsc.VectorSubcoreMesh(core_axis_name="c", subcore_axis_name="s")

    @functools.partial(
        pl.kernel,
        mesh=mesh,
        compiler_params=pltpu.CompilerParams(needs_layout_passes=False),
        out_type=[jax.ShapeDtypeStruct((TOT,), jnp.float32)] * 6,
        scratch_types=[
            pltpu.VMEM((NTAB,), jnp.float32),
            pltpu.VMEM((NTAB,), jnp.float32),
            pltpu.VMEM((NTAB,), jnp.float32),
            pltpu.VMEM((CHUNK,), jnp.int32),
            pltpu.VMEM((CHUNK,), jnp.float32),
            pltpu.VMEM((CHUNK,), jnp.float32),
            pltpu.VMEM((CHUNK,), jnp.float32),
            pltpu.VMEM((CHUNK,), jnp.float32),
            pltpu.VMEM((CHUNK,), jnp.float32),
            pltpu.VMEM((CHUNK,), jnp.float32),
        ],
    )
    def gk(tx_h, ty_h, tz_h, idx_h, ox_h, oy_h, oz_h, dx_h, dy_h, dz_h,
           tx_v, ty_v, tz_v, idx_v, ox_v, oy_v, oz_v, dx_v, dy_v, dz_v):
        wid = lax.axis_index("s") * 2 + lax.axis_index("c")
        base = wid * CHUNK
        pltpu.sync_copy(tx_h, tx_v)
        pltpu.sync_copy(ty_h, ty_v)
        pltpu.sync_copy(tz_h, tz_v)
        pltpu.sync_copy(idx_h.at[pl.ds(base, CHUNK)], idx_v)

        def body(i, _):
            sl = pl.ds(i * 16, 16)
            iv = idx_v[sl]
            pos = base + i * 16 + lax.iota(jnp.int32, 16)
            sv = pos % N + (pos // (N * KNB)) * N
            gx = plsc.load_gather(tx_v, [iv])
            gy = plsc.load_gather(ty_v, [iv])
            gz = plsc.load_gather(tz_v, [iv])
            ox_v[sl] = gx
            oy_v[sl] = gy
            oz_v[sl] = gz
            dx_v[sl] = gx - plsc.load_gather(tx_v, [sv])
            dy_v[sl] = gy - plsc.load_gather(ty_v, [sv])
            dz_v[sl] = gz - plsc.load_gather(tz_v, [sv])
            return 0

        lax.fori_loop(0, CHUNK // 16, body, 0)
        pltpu.sync_copy(ox_v, ox_h.at[pl.ds(base, CHUNK)])
        pltpu.sync_copy(oy_v, oy_h.at[pl.ds(base, CHUNK)])
        pltpu.sync_copy(oz_v, oz_h.at[pl.ds(base, CHUNK)])
        pltpu.sync_copy(dx_v, dx_h.at[pl.ds(base, CHUNK)])
        pltpu.sync_copy(dy_v, dy_h.at[pl.ds(base, CHUNK)])
        pltpu.sync_copy(dz_v, dz_h.at[pl.ds(base, CHUNK)])

    outs = gk(tx, ty, tz, idx_flat)
    return [o.reshape(B, KNB, N) for o in outs]


def _knn_body(cols_ref, rows_ref, out_ref):
    b = pl.program_id(0)
    rows = rows_ref[0]
    x0 = cols_ref[0, 0:1, :]
    x1 = cols_ref[0, 1:2, :]
    x2 = cols_ref[0, 2:3, :]
    r0 = rows[:, 0:1]
    r1 = rows[:, 1:2]
    r2 = rows[:, 2:3]
    rr = r0 * r0 + r1 * r1 + r2 * r2
    cc = x0 * x0 + x1 * x1 + x2 * x2
    dot = (_q(r0) * _q(x0) + _q(r1) * _q(x1) + _q(r2) * _q(x2))
    inner = -2.0 * dot
    s = (-rr - inner) - cc
    iota = lax.broadcasted_iota(jnp.int32, (R, N), 1)
    for j in range(KNB):
        m = jnp.max(s, axis=1, keepdims=True)
        cand = jnp.where(s >= m, iota, N)
        idx = jnp.min(cand, axis=1, keepdims=True)
        out_ref[0, :, j:j + 1] = idx + b * N
        s = jnp.where(iota == idx, NEG, s)


def _knn(xp, xt):
    return pl.pallas_call(
        _knn_body,
        grid=(B, N // R),
        in_specs=[
            pl.BlockSpec((1, 3, N), lambda b, i: (b, 0, 0)),
            pl.BlockSpec((1, R, 3), lambda b, i: (b, i, 0)),
        ],
        out_specs=pl.BlockSpec((1, R, KNB), lambda b, i: (b, i, 0)),
        out_shape=jax.ShapeDtypeStruct((B, N, KNB), jnp.int32),
        interpret=False,
    )(xp, xt)


def _acos(x):
    ax = jnp.abs(x)
    t = jnp.sqrt(jnp.maximum(1.0 - ax, 0.0))
    p = t * (1.5707288 + ax * (-0.2121144 + ax * (0.0742610 + ax * (-0.0187293))))
    return jnp.where(x >= 0.0, p, math.pi - p)


def _cov_eig_body(dx_ref, dy_ref, dz_ref, out_ref):
    dx = _q(dx_ref[0])
    dy = _q(dy_ref[0])
    dz = _q(dz_ref[0])
    a00 = jnp.sum(dx * dx, axis=0, keepdims=True)
    a11 = jnp.sum(dy * dy, axis=0, keepdims=True)
    a22 = jnp.sum(dz * dz, axis=0, keepdims=True)
    a01 = jnp.sum(dx * dy, axis=0, keepdims=True)
    a02 = jnp.sum(dx * dz, axis=0, keepdims=True)
    a12 = jnp.sum(dy * dz, axis=0, keepdims=True)

    q = (a00 + a11 + a22) / 3.0
    p1 = a01 * a01 + a02 * a02 + a12 * a12
    b00 = a00 - q
    b11 = a11 - q
    b22 = a22 - q
    p2 = b00 * b00 + b11 * b11 + b22 * b22 + 2.0 * p1
    p = jnp.sqrt(jnp.maximum(p2 / 6.0, 0.0))
    pinv = jnp.where(p > 1e-20, 1.0 / jnp.maximum(p, 1e-30), 0.0)
    c00 = b00 * pinv
    c11 = b11 * pinv
    c22 = b22 * pinv
    c01 = a01 * pinv
    c02 = a02 * pinv
    c12 = a12 * pinv
    det = (c00 * (c11 * c22 - c12 * c12)
           - c01 * (c01 * c22 - c12 * c02)
           + c02 * (c01 * c12 - c11 * c02))
    r = jnp.clip(det * 0.5, -1.0, 1.0)
    phi = _acos(r) / 3.0
    c1 = jnp.cos(phi)
    c3 = jnp.cos(phi + 2.0 * math.pi / 3.0)
    for _ in range(2):
        d1 = 12.0 * c1 * c1 - 3.0
        ok1 = jnp.abs(d1) > 1e-3
        c1 = c1 - jnp.where(
            ok1, (4.0 * c1 * c1 * c1 - 3.0 * c1 - r)
            / jnp.where(ok1, d1, 1.0), 0.0)
        d3 = 12.0 * c3 * c3 - 3.0
        ok3 = jnp.abs(d3) > 1e-3
        c3 = c3 - jnp.where(
            ok3, (4.0 * c3 * c3 * c3 - 3.0 * c3 - r)
            / jnp.where(ok3, d3, 1.0), 0.0)
    e1 = q + 2.0 * p * c1
    e3 = q + 2.0 * p * c3
    e2 = 3.0 * q - e1 - e3
    out_ref[0, 0:1, :] = e3
    out_ref[0, 1:2, :] = e2
    out_ref[0, 2:3, :] = e1


def _cov_eig(dx, dy, dz):
    plane = pl.BlockSpec((1, KNB, TN), lambda b, i: (b, 0, i))
    return pl.pallas_call(
        _cov_eig_body,
        grid=(B, N // TN),
        in_specs=[plane, plane, plane],
        out_specs=pl.BlockSpec((1, 3, TN), lambda b, i: (b, 0, i)),
        out_shape=jax.ShapeDtypeStruct((B, 3, N), jnp.float32),
        interpret=False,
    )(dx, dy, dz)


def _feat(j, dx, dy, dz, nx, ny, nz, gx, gy, gz, fx, fy, fz, dist):
    return jnp.concatenate(
        [dx[j:j + 1], dy[j:j + 1], dz[j:j + 1],
         nx[j:j + 1], ny[j:j + 1], nz[j:j + 1],
         gx[j:j + 1], gy[j:j + 1], gz[j:j + 1],
         fx[j:j + 1], fy[j:j + 1], fz[j:j + 1],
         dist[j:j + 1]], axis=0)


def _load_planes(refs):
    return [r[0] for r in refs]


def _mom_body(dx_ref, dy_ref, dz_ref, nx_ref, ny_ref, nz_ref,
              gx_ref, gy_ref, gz_ref, fx_ref, fy_ref, fz_ref, g_ref):
    b = pl.program_id(0)
    i = pl.program_id(1)

    @pl.when(jnp.logical_and(b == 0, i == 0))
    def _():
        g_ref[...] = jnp.zeros((16, 16), jnp.float32)

    dx, dy, dz, nx, ny, nz, gx, gy, gz, fx, fy, fz = _load_planes(
        [dx_ref, dy_ref, dz_ref, nx_ref, ny_ref, nz_ref,
         gx_ref, gy_ref, gz_ref, fx_ref, fy_ref, fz_ref])
    dist = jnp.sqrt(dx * dx + dy * dy + dz * dz + 1e-12)
    ones = jnp.ones((1, TN), jnp.float32)
    zeros = jnp.zeros((2, TN), jnp.float32)
    acc = jnp.zeros((16, 16), jnp.float32)
    for j in range(KNB):
        f = _q(_feat(j, dx, dy, dz, nx, ny, nz, gx, gy, gz, fx, fy, fz,
                     dist))
        f16 = jnp.concatenate([f, ones, zeros], axis=0)
        acc = acc + lax.dot_general(
            f16, f16, (((1,), (1,)), ((), ())),
            preferred_element_type=jnp.float32)
    g_ref[...] += acc


def _moments(planes):
    plane = pl.BlockSpec((1, KNB, TN), lambda b, i: (b, 0, i))
    return pl.pallas_call(
        _mom_body,
        grid=(B, N // TN),
        in_specs=[plane] * 12,
        out_specs=pl.BlockSpec((16, 16), lambda b, i: (0, 0)),
        out_shape=jax.ShapeDtypeStruct((16, 16), jnp.float32),
        interpret=False,
    )(*planes)


def _final_body(dx_ref, dy_ref, dz_ref, nx_ref, ny_ref, nz_ref,
                gx_ref, gy_ref, gz_ref, fx_ref, fy_ref, fz_ref,
                g_ref, w_ref, gm_ref, bt_ref, out_ref):
    g = g_ref[...]
    cnt = g[13:14, 13:14]
    wq = _q(w_ref[...])
    gm = gm_ref[...]
    bt = bt_ref[...]
    inv_cnt = 1.0 / cnt[0, 0]
    m1 = g[0:13, 13:14] * inv_cnt
    m2 = g[0:13, 0:13] * inv_cnt
    mean_y = lax.dot_general(wq, m1, (((1,), (0,)), ((), ())),
                             preferred_element_type=jnp.float32)
    b1 = lax.dot_general(wq, m2, (((1,), (0,)), ((), ())),
                         preferred_element_type=jnp.float32)
    e2 = jnp.sum(wq * b1, axis=1, keepdims=True)
    var = jnp.maximum(e2 - mean_y * mean_y, 0.0)
    scale = gm / jnp.sqrt(var + 1e-5)
    shift = bt - mean_y * scale

    dx, dy, dz, nx, ny, nz, gx, gy, gz, fx, fy, fz = _load_planes(
        [dx_ref, dy_ref, dz_ref, nx_ref, ny_ref, nz_ref,
         gx_ref, gy_ref, gz_ref, fx_ref, fy_ref, fz_ref])
    dist = jnp.sqrt(dx * dx + dy * dy + dz * dz + 1e-12)
    acc = jnp.full((64, TN), NEG, jnp.float32)
    for j in range(KNB):
        f = _q(_feat(j, dx, dy, dz, nx, ny, nz, gx, gy, gz, fx, fy, fz,
                     dist))
        y = lax.dot_general(wq, f, (((1,), (0,)), ((), ())),
                            preferred_element_type=jnp.float32)
        y = y * scale + shift
        y = jnp.where(y >= 0.0, y, 0.2 * y)
        acc = jnp.maximum(acc, y)
    out_ref[0] = acc


def _final(planes, g, w, gm, bt):
    plane = pl.BlockSpec((1, KNB, TN), lambda b, i: (b, 0, i))
    const = lambda shape: pl.BlockSpec(shape, lambda b, i: (0, 0))
    return pl.pallas_call(
        _final_body,
        grid=(B, N // TN),
        in_specs=[plane] * 12 + [const((16, 16)), const((64, 13)),
                                 const((64, 1)), const((64, 1))],
        out_specs=pl.BlockSpec((1, 64, TN), lambda b, i: (b, 0, i)),
        out_shape=jax.ShapeDtypeStruct((B, 64, N), jnp.float32),
        interpret=False,
    )(*planes, g, w, gm, bt)


def kernel(x, W, gamma, beta):
    xt = jnp.transpose(x, (0, 2, 1))

    idx_eu = _knn(x, xt)
    idx_eu_t = jnp.transpose(idx_eu, (0, 2, 1)).reshape(-1)
    nx, ny, nz, dx, dy, dz = _sc_gather6(
        x[:, 0, :].reshape(-1), x[:, 1, :].reshape(-1),
        x[:, 2, :].reshape(-1), idx_eu_t)

    ev = _cov_eig(dx, dy, dz)
    evt = jnp.transpose(ev, (0, 2, 1))
    idx_ei = _knn(ev, evt)
    idx_ei_t = jnp.transpose(idx_ei, (0, 2, 1)).reshape(-1)
    fx, fy, fz, gx, gy, gz = _sc_gather6(
        ev[:, 0, :].reshape(-1), ev[:, 1, :].reshape(-1),
        ev[:, 2, :].reshape(-1), idx_ei_t)

    planes = [dx, dy, dz, nx, ny, nz, gx, gy, gz, fx, fy, fz]
    g = _moments(planes)
    return _final(planes, g, W, gamma.reshape(64, 1), beta.reshape(64, 1))


# speedup vs baseline: 2.3632x; 1.2105x over previous
import functools
import math

import jax
import jax.numpy as jnp
from jax import lax
from jax.experimental import pallas as pl
from jax.experimental.pallas import tpu as pltpu
from jax.experimental.pallas import tpu_sc as plsc

B = 8
N = 2048
KNB = 20
R = 512
TN = 512
TOT = B * N * KNB
NW = 32
CHUNK = TOT // NW
NTAB = B * N
NEG = -3.0e38


def _q(v):
    b = jax.lax.bitcast_convert_type(v, jnp.uint32)
    lsb = (b >> jnp.uint32(16)) & jnp.uint32(1)
    r = (b + jnp.uint32(0x7FFF) + lsb) & jnp.uint32(0xFFFF0000)
    return jax.lax.bitcast_convert_type(r, jnp.float32)


def _sc_gather6(tx, ty, tz, idx_flat):
    mesh = plsc.VectorSubcoreMesh(core_axis_name="c", subcore_axis_name="s")

    @functools.partial(
        pl.kernel,
        mesh=mesh,
        compiler_params=pltpu.CompilerParams(needs_layout_passes=False),
        out_type=[jax.ShapeDtypeStruct((TOT,), jnp.float32)] * 6,
        scratch_types=[
            pltpu.VMEM((NTAB,), jnp.float32),
            pltpu.VMEM((NTAB,), jnp.float32),
            pltpu.VMEM((NTAB,), jnp.float32),
            pltpu.VMEM((CHUNK,), jnp.int32),
            pltpu.VMEM((CHUNK,), jnp.float32),
            pltpu.VMEM((CHUNK,), jnp.float32),
            pltpu.VMEM((CHUNK,), jnp.float32),
            pltpu.VMEM((CHUNK,), jnp.float32),
            pltpu.VMEM((CHUNK,), jnp.float32),
            pltpu.VMEM((CHUNK,), jnp.float32),
        ],
    )
    def gk(tx_h, ty_h, tz_h, idx_h, ox_h, oy_h, oz_h, dx_h, dy_h, dz_h,
           tx_v, ty_v, tz_v, idx_v, ox_v, oy_v, oz_v, dx_v, dy_v, dz_v):
        wid = lax.axis_index("s") * 2 + lax.axis_index("c")
        base = wid * CHUNK
        pltpu.sync_copy(tx_h, tx_v)
        pltpu.sync_copy(ty_h, ty_v)
        pltpu.sync_copy(tz_h, tz_v)
        pltpu.sync_copy(idx_h.at[pl.ds(base, CHUNK)], idx_v)

        def body(i, _):
            sl = pl.ds(i * 16, 16)
            iv = idx_v[sl]
            pos = base + i * 16 + lax.iota(jnp.int32, 16)
            sv = pos % N + (pos // (N * KNB)) * N
            gx = plsc.load_gather(tx_v, [iv])
            gy = plsc.load_gather(ty_v, [iv])
            gz = plsc.load_gather(tz_v, [iv])
            ox_v[sl] = gx
            oy_v[sl] = gy
            oz_v[sl] = gz
            dx_v[sl] = gx - plsc.load_gather(tx_v, [sv])
            dy_v[sl] = gy - plsc.load_gather(ty_v, [sv])
            dz_v[sl] = gz - plsc.load_gather(tz_v, [sv])
            return 0

        lax.fori_loop(0, CHUNK // 16, body, 0)
        pltpu.sync_copy(ox_v, ox_h.at[pl.ds(base, CHUNK)])
        pltpu.sync_copy(oy_v, oy_h.at[pl.ds(base, CHUNK)])
        pltpu.sync_copy(oz_v, oz_h.at[pl.ds(base, CHUNK)])
        pltpu.sync_copy(dx_v, dx_h.at[pl.ds(base, CHUNK)])
        pltpu.sync_copy(dy_v, dy_h.at[pl.ds(base, CHUNK)])
        pltpu.sync_copy(dz_v, dz_h.at[pl.ds(base, CHUNK)])

    outs = gk(tx, ty, tz, idx_flat)
    return [o.reshape(B, KNB, N) for o in outs]


def _knn_body(cols_ref, rows_ref, out_ref):
    b = pl.program_id(0)
    rows = rows_ref[0]
    x0 = cols_ref[0, 0:1, :]
    x1 = cols_ref[0, 1:2, :]
    x2 = cols_ref[0, 2:3, :]
    r0 = rows[:, 0:1]
    r1 = rows[:, 1:2]
    r2 = rows[:, 2:3]
    rr = r0 * r0 + r1 * r1 + r2 * r2
    cc = x0 * x0 + x1 * x1 + x2 * x2
    dot = (_q(r0) * _q(x0) + _q(r1) * _q(x1) + _q(r2) * _q(x2))
    inner = -2.0 * dot
    s = (-rr - inner) - cc
    iota = lax.broadcasted_iota(jnp.int32, (R, N), 1)
    for j in range(KNB):
        idx = jnp.argmax(s, axis=1).reshape(R, 1)
        out_ref[0, :, j:j + 1] = idx + b * N
        s = jnp.where(iota == idx, NEG, s)


def _knn(xp, xt):
    return pl.pallas_call(
        _knn_body,
        grid=(B, N // R),
        in_specs=[
            pl.BlockSpec((1, 3, N), lambda b, i: (b, 0, 0)),
            pl.BlockSpec((1, R, 3), lambda b, i: (b, i, 0)),
        ],
        out_specs=pl.BlockSpec((1, R, KNB), lambda b, i: (b, i, 0)),
        out_shape=jax.ShapeDtypeStruct((B, N, KNB), jnp.int32),
        interpret=False,
    )(xp, xt)


def _acos(x):
    ax = jnp.abs(x)
    t = jnp.sqrt(jnp.maximum(1.0 - ax, 0.0))
    p = t * (1.5707288 + ax * (-0.2121144 + ax * (0.0742610 + ax * (-0.0187293))))
    return jnp.where(x >= 0.0, p, math.pi - p)


def _cov_eig_body(dx_ref, dy_ref, dz_ref, out_ref):
    dx = _q(dx_ref[0])
    dy = _q(dy_ref[0])
    dz = _q(dz_ref[0])
    a00 = jnp.sum(dx * dx, axis=0, keepdims=True)
    a11 = jnp.sum(dy * dy, axis=0, keepdims=True)
    a22 = jnp.sum(dz * dz, axis=0, keepdims=True)
    a01 = jnp.sum(dx * dy, axis=0, keepdims=True)
    a02 = jnp.sum(dx * dz, axis=0, keepdims=True)
    a12 = jnp.sum(dy * dz, axis=0, keepdims=True)

    q = (a00 + a11 + a22) / 3.0
    p1 = a01 * a01 + a02 * a02 + a12 * a12
    b00 = a00 - q
    b11 = a11 - q
    b22 = a22 - q
    p2 = b00 * b00 + b11 * b11 + b22 * b22 + 2.0 * p1
    p = jnp.sqrt(jnp.maximum(p2 / 6.0, 0.0))
    pinv = jnp.where(p > 1e-20, 1.0 / jnp.maximum(p, 1e-30), 0.0)
    c00 = b00 * pinv
    c11 = b11 * pinv
    c22 = b22 * pinv
    c01 = a01 * pinv
    c02 = a02 * pinv
    c12 = a12 * pinv
    det = (c00 * (c11 * c22 - c12 * c12)
           - c01 * (c01 * c22 - c12 * c02)
           + c02 * (c01 * c12 - c11 * c02))
    r = jnp.clip(det * 0.5, -1.0, 1.0)
    phi = _acos(r) / 3.0
    c1 = jnp.cos(phi)
    c3 = jnp.cos(phi + 2.0 * math.pi / 3.0)
    for _ in range(2):
        d1 = 12.0 * c1 * c1 - 3.0
        ok1 = jnp.abs(d1) > 1e-3
        c1 = c1 - jnp.where(
            ok1, (4.0 * c1 * c1 * c1 - 3.0 * c1 - r)
            / jnp.where(ok1, d1, 1.0), 0.0)
        d3 = 12.0 * c3 * c3 - 3.0
        ok3 = jnp.abs(d3) > 1e-3
        c3 = c3 - jnp.where(
            ok3, (4.0 * c3 * c3 * c3 - 3.0 * c3 - r)
            / jnp.where(ok3, d3, 1.0), 0.0)
    e1 = q + 2.0 * p * c1
    e3 = q + 2.0 * p * c3
    e2 = 3.0 * q - e1 - e3
    out_ref[0, 0:1, :] = e3
    out_ref[0, 1:2, :] = e2
    out_ref[0, 2:3, :] = e1


def _cov_eig(dx, dy, dz):
    plane = pl.BlockSpec((1, KNB, TN), lambda b, i: (b, 0, i))
    return pl.pallas_call(
        _cov_eig_body,
        grid=(B, N // TN),
        in_specs=[plane, plane, plane],
        out_specs=pl.BlockSpec((1, 3, TN), lambda b, i: (b, 0, i)),
        out_shape=jax.ShapeDtypeStruct((B, 3, N), jnp.float32),
        interpret=False,
    )(dx, dy, dz)


def _feat(j, dx, dy, dz, nx, ny, nz, gx, gy, gz, fx, fy, fz, dist):
    return jnp.concatenate(
        [dx[j:j + 1], dy[j:j + 1], dz[j:j + 1],
         nx[j:j + 1], ny[j:j + 1], nz[j:j + 1],
         gx[j:j + 1], gy[j:j + 1], gz[j:j + 1],
         fx[j:j + 1], fy[j:j + 1], fz[j:j + 1],
         dist[j:j + 1]], axis=0)


def _load_planes(refs):
    return [r[0] for r in refs]


def _mom_body(dx_ref, dy_ref, dz_ref, nx_ref, ny_ref, nz_ref,
              gx_ref, gy_ref, gz_ref, fx_ref, fy_ref, fz_ref, g_ref):
    b = pl.program_id(0)
    i = pl.program_id(1)

    @pl.when(jnp.logical_and(b == 0, i == 0))
    def _():
        g_ref[...] = jnp.zeros((16, 16), jnp.float32)

    dx, dy, dz, nx, ny, nz, gx, gy, gz, fx, fy, fz = _load_planes(
        [dx_ref, dy_ref, dz_ref, nx_ref, ny_ref, nz_ref,
         gx_ref, gy_ref, gz_ref, fx_ref, fy_ref, fz_ref])
    dist = jnp.sqrt(dx * dx + dy * dy + dz * dz + 1e-12)
    ones = jnp.ones((1, TN), jnp.float32)
    zeros = jnp.zeros((2, TN), jnp.float32)
    acc = jnp.zeros((16, 16), jnp.float32)
    for j in range(KNB):
        f = _q(_feat(j, dx, dy, dz, nx, ny, nz, gx, gy, gz, fx, fy, fz,
                     dist))
        f16 = jnp.concatenate([f, ones, zeros], axis=0)
        acc = acc + lax.dot_general(
            f16, f16, (((1,), (1,)), ((), ())),
            preferred_element_type=jnp.float32)
    g_ref[...] += acc


def _moments(planes):
    plane = pl.BlockSpec((1, KNB, TN), lambda b, i: (b, 0, i))
    return pl.pallas_call(
        _mom_body,
        grid=(B, N // TN),
        in_specs=[plane] * 12,
        out_specs=pl.BlockSpec((16, 16), lambda b, i: (0, 0)),
        out_shape=jax.ShapeDtypeStruct((16, 16), jnp.float32),
        interpret=False,
    )(*planes)


def _final_body(dx_ref, dy_ref, dz_ref, nx_ref, ny_ref, nz_ref,
                gx_ref, gy_ref, gz_ref, fx_ref, fy_ref, fz_ref,
                g_ref, w_ref, gm_ref, bt_ref, out_ref):
    g = g_ref[...]
    cnt = g[13:14, 13:14]
    wq = _q(w_ref[...])
    gm = gm_ref[...]
    bt = bt_ref[...]
    inv_cnt = 1.0 / cnt[0, 0]
    m1 = g[0:13, 13:14] * inv_cnt
    m2 = g[0:13, 0:13] * inv_cnt
    mean_y = lax.dot_general(wq, m1, (((1,), (0,)), ((), ())),
                             preferred_element_type=jnp.float32)
    b1 = lax.dot_general(wq, m2, (((1,), (0,)), ((), ())),
                         preferred_element_type=jnp.float32)
    e2 = jnp.sum(wq * b1, axis=1, keepdims=True)
    var = jnp.maximum(e2 - mean_y * mean_y, 0.0)
    scale = gm / jnp.sqrt(var + 1e-5)
    shift = bt - mean_y * scale

    dx, dy, dz, nx, ny, nz, gx, gy, gz, fx, fy, fz = _load_planes(
        [dx_ref, dy_ref, dz_ref, nx_ref, ny_ref, nz_ref,
         gx_ref, gy_ref, gz_ref, fx_ref, fy_ref, fz_ref])
    dist = jnp.sqrt(dx * dx + dy * dy + dz * dz + 1e-12)
    acc = jnp.full((64, TN), NEG, jnp.float32)
    for j in range(KNB):
        f = _q(_feat(j, dx, dy, dz, nx, ny, nz, gx, gy, gz, fx, fy, fz,
                     dist))
        y = lax.dot_general(wq, f, (((1,), (0,)), ((), ())),
                            preferred_element_type=jnp.float32)
        y = y * scale + shift
        y = jnp.where(y >= 0.0, y, 0.2 * y)
        acc = jnp.maximum(acc, y)
    out_ref[0] = acc


def _final(planes, g, w, gm, bt):
    plane = pl.BlockSpec((1, KNB, TN), lambda b, i: (b, 0, i))
    const = lambda shape: pl.BlockSpec(shape, lambda b, i: (0, 0))
    return pl.pallas_call(
        _final_body,
        grid=(B, N // TN),
        in_specs=[plane] * 12 + [const((16, 16)), const((64, 13)),
                                 const((64, 1)), const((64, 1))],
        out_specs=pl.BlockSpec((1, 64, TN), lambda b, i: (b, 0, i)),
        out_shape=jax.ShapeDtypeStruct((B, 64, N), jnp.float32),
        interpret=False,
    )(*planes, g, w, gm, bt)


def kernel(x, W, gamma, beta):
    xt = jnp.transpose(x, (0, 2, 1))

    idx_eu = _knn(x, xt)
    idx_eu_t = jnp.transpose(idx_eu, (0, 2, 1)).reshape(-1)
    nx, ny, nz, dx, dy, dz = _sc_gather6(
        x[:, 0, :].reshape(-1), x[:, 1, :].reshape(-1),
        x[:, 2, :].reshape(-1), idx_eu_t)

    ev = _cov_eig(dx, dy, dz)
    evt = jnp.transpose(ev, (0, 2, 1))
    idx_ei = _knn(ev, evt)
    idx_ei_t = jnp.transpose(idx_ei, (0, 2, 1)).reshape(-1)
    fx, fy, fz, gx, gy, gz = _sc_gather6(
        ev[:, 0, :].reshape(-1), ev[:, 1, :].reshape(-1),
        ev[:, 2, :].reshape(-1), idx_ei_t)

    planes = [dx, dy, dz, nx, ny, nz, gx, gy, gz, fx, fy, fz]
    g = _moments(planes)
    return _final(planes, g, W, gamma.reshape(64, 1), beta.reshape(64, 1))


# speedup vs baseline: 2.3635x; 1.0001x over previous
import functools
import math

import jax
import jax.numpy as jnp
from jax import lax
from jax.experimental import pallas as pl
from jax.experimental.pallas import tpu as pltpu
from jax.experimental.pallas import tpu_sc as plsc

B = 8
N = 2048
KNB = 20
R = 512
TN = 512
TOT = B * N * KNB
NW = 32
CHUNK = TOT // NW
NTAB = B * N
NEG = -3.0e38


def _q(v):
    b = jax.lax.bitcast_convert_type(v, jnp.uint32)
    lsb = (b >> jnp.uint32(16)) & jnp.uint32(1)
    r = (b + jnp.uint32(0x7FFF) + lsb) & jnp.uint32(0xFFFF0000)
    return jax.lax.bitcast_convert_type(r, jnp.float32)


def _sc_gather6(tx, ty, tz, idx_flat):
    mesh = plsc.VectorSubcoreMesh(core_axis_name="c", subcore_axis_name="s")

    @functools.partial(
        pl.kernel,
        mesh=mesh,
        compiler_params=pltpu.CompilerParams(needs_layout_passes=False),
        out_type=[jax.ShapeDtypeStruct((TOT,), jnp.float32)] * 6,
        scratch_types=[
            pltpu.VMEM((NTAB,), jnp.float32),
            pltpu.VMEM((NTAB,), jnp.float32),
            pltpu.VMEM((NTAB,), jnp.float32),
            pltpu.VMEM((CHUNK,), jnp.int32),
            pltpu.VMEM((CHUNK,), jnp.float32),
            pltpu.VMEM((CHUNK,), jnp.float32),
            pltpu.VMEM((CHUNK,), jnp.float32),
            pltpu.VMEM((CHUNK,), jnp.float32),
            pltpu.VMEM((CHUNK,), jnp.float32),
            pltpu.VMEM((CHUNK,), jnp.float32),
        ],
    )
    def gk(tx_h, ty_h, tz_h, idx_h, ox_h, oy_h, oz_h, dx_h, dy_h, dz_h,
           tx_v, ty_v, tz_v, idx_v, ox_v, oy_v, oz_v, dx_v, dy_v, dz_v):
        wid = lax.axis_index("s") * 2 + lax.axis_index("c")
        base = wid * CHUNK
        pltpu.sync_copy(tx_h, tx_v)
        pltpu.sync_copy(ty_h, ty_v)
        pltpu.sync_copy(tz_h, tz_v)
        pltpu.sync_copy(idx_h.at[pl.ds(base, CHUNK)], idx_v)

        def body(i, _):
            sl = pl.ds(i * 16, 16)
            iv = idx_v[sl]
            pos = base + i * 16 + lax.iota(jnp.int32, 16)
            sv = pos % N + (pos // (N * KNB)) * N
            gx = plsc.load_gather(tx_v, [iv])
            gy = plsc.load_gather(ty_v, [iv])
            gz = plsc.load_gather(tz_v, [iv])
            ox_v[sl] = gx
            oy_v[sl] = gy
            oz_v[sl] = gz
            dx_v[sl] = gx - plsc.load_gather(tx_v, [sv])
            dy_v[sl] = gy - plsc.load_gather(ty_v, [sv])
            dz_v[sl] = gz - plsc.load_gather(tz_v, [sv])
            return 0

        lax.fori_loop(0, CHUNK // 16, body, 0)
        pltpu.sync_copy(ox_v, ox_h.at[pl.ds(base, CHUNK)])
        pltpu.sync_copy(oy_v, oy_h.at[pl.ds(base, CHUNK)])
        pltpu.sync_copy(oz_v, oz_h.at[pl.ds(base, CHUNK)])
        pltpu.sync_copy(dx_v, dx_h.at[pl.ds(base, CHUNK)])
        pltpu.sync_copy(dy_v, dy_h.at[pl.ds(base, CHUNK)])
        pltpu.sync_copy(dz_v, dz_h.at[pl.ds(base, CHUNK)])

    outs = gk(tx, ty, tz, idx_flat)
    return [o.reshape(B, KNB, N) for o in outs]


def _knn_body(cols_ref, rows_ref, out_ref):
    b = pl.program_id(0)
    rows = rows_ref[0]
    x0 = cols_ref[0, 0:1, :]
    x1 = cols_ref[0, 1:2, :]
    x2 = cols_ref[0, 2:3, :]
    r0 = rows[:, 0:1]
    r1 = rows[:, 1:2]
    r2 = rows[:, 2:3]
    rr = r0 * r0 + r1 * r1 + r2 * r2
    cc = x0 * x0 + x1 * x1 + x2 * x2
    dot = (_q(r0) * _q(x0) + _q(r1) * _q(x1) + _q(r2) * _q(x2))
    inner = -2.0 * dot
    s = (-rr - inner) - cc
    iota = lax.broadcasted_iota(jnp.int32, (R, N), 1)
    for j in range(KNB):
        idx = jnp.argmax(s, axis=1).reshape(R, 1)
        out_ref[0, :, j:j + 1] = idx + b * N
        if j < KNB - 1:
            s = jnp.where(iota == idx, NEG, s)


def _knn(xp, xt):
    return pl.pallas_call(
        _knn_body,
        grid=(B, N // R),
        in_specs=[
            pl.BlockSpec((1, 3, N), lambda b, i: (b, 0, 0)),
            pl.BlockSpec((1, R, 3), lambda b, i: (b, i, 0)),
        ],
        out_specs=pl.BlockSpec((1, R, KNB), lambda b, i: (b, i, 0)),
        out_shape=jax.ShapeDtypeStruct((B, N, KNB), jnp.int32),
        interpret=False,
    )(xp, xt)


def _acos(x):
    ax = jnp.abs(x)
    t = jnp.sqrt(jnp.maximum(1.0 - ax, 0.0))
    p = t * (1.5707288 + ax * (-0.2121144 + ax * (0.0742610 + ax * (-0.0187293))))
    return jnp.where(x >= 0.0, p, math.pi - p)


def _cov_eig_body(dx_ref, dy_ref, dz_ref, out_ref):
    dx = _q(dx_ref[0])
    dy = _q(dy_ref[0])
    dz = _q(dz_ref[0])
    a00 = jnp.sum(dx * dx, axis=0, keepdims=True)
    a11 = jnp.sum(dy * dy, axis=0, keepdims=True)
    a22 = jnp.sum(dz * dz, axis=0, keepdims=True)
    a01 = jnp.sum(dx * dy, axis=0, keepdims=True)
    a02 = jnp.sum(dx * dz, axis=0, keepdims=True)
    a12 = jnp.sum(dy * dz, axis=0, keepdims=True)

    q = (a00 + a11 + a22) / 3.0
    p1 = a01 * a01 + a02 * a02 + a12 * a12
    b00 = a00 - q
    b11 = a11 - q
    b22 = a22 - q
    p2 = b00 * b00 + b11 * b11 + b22 * b22 + 2.0 * p1
    p = jnp.sqrt(jnp.maximum(p2 / 6.0, 0.0))
    pinv = jnp.where(p > 1e-20, 1.0 / jnp.maximum(p, 1e-30), 0.0)
    c00 = b00 * pinv
    c11 = b11 * pinv
    c22 = b22 * pinv
    c01 = a01 * pinv
    c02 = a02 * pinv
    c12 = a12 * pinv
    det = (c00 * (c11 * c22 - c12 * c12)
           - c01 * (c01 * c22 - c12 * c02)
           + c02 * (c01 * c12 - c11 * c02))
    r = jnp.clip(det * 0.5, -1.0, 1.0)
    phi = _acos(r) / 3.0
    c1 = jnp.cos(phi)
    c3 = jnp.cos(phi + 2.0 * math.pi / 3.0)
    for _ in range(2):
        d1 = 12.0 * c1 * c1 - 3.0
        ok1 = jnp.abs(d1) > 1e-3
        c1 = c1 - jnp.where(
            ok1, (4.0 * c1 * c1 * c1 - 3.0 * c1 - r)
            / jnp.where(ok1, d1, 1.0), 0.0)
        d3 = 12.0 * c3 * c3 - 3.0
        ok3 = jnp.abs(d3) > 1e-3
        c3 = c3 - jnp.where(
            ok3, (4.0 * c3 * c3 * c3 - 3.0 * c3 - r)
            / jnp.where(ok3, d3, 1.0), 0.0)
    e1 = q + 2.0 * p * c1
    e3 = q + 2.0 * p * c3
    e2 = 3.0 * q - e1 - e3
    out_ref[0, 0:1, :] = e3
    out_ref[0, 1:2, :] = e2
    out_ref[0, 2:3, :] = e1


def _cov_eig(dx, dy, dz):
    plane = pl.BlockSpec((1, KNB, TN), lambda b, i: (b, 0, i))
    return pl.pallas_call(
        _cov_eig_body,
        grid=(B, N // TN),
        in_specs=[plane, plane, plane],
        out_specs=pl.BlockSpec((1, 3, TN), lambda b, i: (b, 0, i)),
        out_shape=jax.ShapeDtypeStruct((B, 3, N), jnp.float32),
        interpret=False,
    )(dx, dy, dz)


def _feat(j, dx, dy, dz, nx, ny, nz, gx, gy, gz, fx, fy, fz, dist):
    return jnp.concatenate(
        [dx[j:j + 1], dy[j:j + 1], dz[j:j + 1],
         nx[j:j + 1], ny[j:j + 1], nz[j:j + 1],
         gx[j:j + 1], gy[j:j + 1], gz[j:j + 1],
         fx[j:j + 1], fy[j:j + 1], fz[j:j + 1],
         dist[j:j + 1]], axis=0)


def _load_planes(refs):
    return [r[0] for r in refs]


def _mom_body(dx_ref, dy_ref, dz_ref, nx_ref, ny_ref, nz_ref,
              gx_ref, gy_ref, gz_ref, fx_ref, fy_ref, fz_ref, g_ref):
    b = pl.program_id(0)
    i = pl.program_id(1)

    @pl.when(jnp.logical_and(b == 0, i == 0))
    def _():
        g_ref[...] = jnp.zeros((16, 16), jnp.float32)

    dx, dy, dz, nx, ny, nz, gx, gy, gz, fx, fy, fz = _load_planes(
        [dx_ref, dy_ref, dz_ref, nx_ref, ny_ref, nz_ref,
         gx_ref, gy_ref, gz_ref, fx_ref, fy_ref, fz_ref])
    dist = jnp.sqrt(dx * dx + dy * dy + dz * dz + 1e-12)
    ones = jnp.ones((1, TN), jnp.float32)
    zeros = jnp.zeros((2, TN), jnp.float32)
    acc = jnp.zeros((16, 16), jnp.float32)
    for j in range(KNB):
        f = _q(_feat(j, dx, dy, dz, nx, ny, nz, gx, gy, gz, fx, fy, fz,
                     dist))
        f16 = jnp.concatenate([f, ones, zeros], axis=0)
        acc = acc + lax.dot_general(
            f16, f16, (((1,), (1,)), ((), ())),
            preferred_element_type=jnp.float32)
    g_ref[...] += acc


def _moments(planes):
    plane = pl.BlockSpec((1, KNB, TN), lambda b, i: (b, 0, i))
    return pl.pallas_call(
        _mom_body,
        grid=(B, N // TN),
        in_specs=[plane] * 12,
        out_specs=pl.BlockSpec((16, 16), lambda b, i: (0, 0)),
        out_shape=jax.ShapeDtypeStruct((16, 16), jnp.float32),
        interpret=False,
    )(*planes)


def _final_body(dx_ref, dy_ref, dz_ref, nx_ref, ny_ref, nz_ref,
                gx_ref, gy_ref, gz_ref, fx_ref, fy_ref, fz_ref,
                g_ref, w_ref, gm_ref, bt_ref, out_ref):
    g = g_ref[...]
    cnt = g[13:14, 13:14]
    wq = _q(w_ref[...])
    gm = gm_ref[...]
    bt = bt_ref[...]
    inv_cnt = 1.0 / cnt[0, 0]
    m1 = g[0:13, 13:14] * inv_cnt
    m2 = g[0:13, 0:13] * inv_cnt
    mean_y = lax.dot_general(wq, m1, (((1,), (0,)), ((), ())),
                             preferred_element_type=jnp.float32)
    b1 = lax.dot_general(wq, m2, (((1,), (0,)), ((), ())),
                         preferred_element_type=jnp.float32)
    e2 = jnp.sum(wq * b1, axis=1, keepdims=True)
    var = jnp.maximum(e2 - mean_y * mean_y, 0.0)
    scale = gm / jnp.sqrt(var + 1e-5)
    shift = bt - mean_y * scale

    dx, dy, dz, nx, ny, nz, gx, gy, gz, fx, fy, fz = _load_planes(
        [dx_ref, dy_ref, dz_ref, nx_ref, ny_ref, nz_ref,
         gx_ref, gy_ref, gz_ref, fx_ref, fy_ref, fz_ref])
    dist = jnp.sqrt(dx * dx + dy * dy + dz * dz + 1e-12)
    acc = jnp.full((64, TN), NEG, jnp.float32)
    for j in range(KNB):
        f = _q(_feat(j, dx, dy, dz, nx, ny, nz, gx, gy, gz, fx, fy, fz,
                     dist))
        y = lax.dot_general(wq, f, (((1,), (0,)), ((), ())),
                            preferred_element_type=jnp.float32)
        y = y * scale + shift
        y = jnp.where(y >= 0.0, y, 0.2 * y)
        acc = jnp.maximum(acc, y)
    out_ref[0] = acc


def _final(planes, g, w, gm, bt):
    plane = pl.BlockSpec((1, KNB, TN), lambda b, i: (b, 0, i))
    const = lambda shape: pl.BlockSpec(shape, lambda b, i: (0, 0))
    return pl.pallas_call(
        _final_body,
        grid=(B, N // TN),
        in_specs=[plane] * 12 + [const((16, 16)), const((64, 13)),
                                 const((64, 1)), const((64, 1))],
        out_specs=pl.BlockSpec((1, 64, TN), lambda b, i: (b, 0, i)),
        out_shape=jax.ShapeDtypeStruct((B, 64, N), jnp.float32),
        interpret=False,
    )(*planes, g, w, gm, bt)


def kernel(x, W, gamma, beta):
    xt = jnp.transpose(x, (0, 2, 1))

    idx_eu = _knn(x, xt)
    idx_eu_t = jnp.transpose(idx_eu, (0, 2, 1)).reshape(-1)
    nx, ny, nz, dx, dy, dz = _sc_gather6(
        x[:, 0, :].reshape(-1), x[:, 1, :].reshape(-1),
        x[:, 2, :].reshape(-1), idx_eu_t)

    ev = _cov_eig(dx, dy, dz)
    evt = jnp.transpose(ev, (0, 2, 1))
    idx_ei = _knn(ev, evt)
    idx_ei_t = jnp.transpose(idx_ei, (0, 2, 1)).reshape(-1)
    fx, fy, fz, gx, gy, gz = _sc_gather6(
        ev[:, 0, :].reshape(-1), ev[:, 1, :].reshape(-1),
        ev[:, 2, :].reshape(-1), idx_ei_t)

    planes = [dx, dy, dz, nx, ny, nz, gx, gy, gz, fx, fy, fz]
    g = _moments(planes)
    return _final(planes, g, W, gamma.reshape(64, 1), beta.reshape(64, 1))
